# baseline (speedup 1.0000x reference)
import functools

import jax
import jax.numpy as jnp
from jax import lax
from jax.experimental import pallas as pl
from jax.experimental.pallas import tpu as pltpu
from jax.experimental.pallas import tpu_sc as plsc

CHUNK = 128


@functools.lru_cache(maxsize=None)
def _make_gather(batch, vocab, dim):
  info = plsc.get_sparse_core_info()
  nc, ns = info.num_cores, info.num_subcores
  nw = nc * ns
  b_per_w = batch // nw
  n_ch = b_per_w // CHUNK
  mesh = plsc.VectorSubcoreMesh(core_axis_name="c", subcore_axis_name="s")

  @functools.partial(
      pl.kernel,
      mesh=mesh,
      out_type=jax.ShapeDtypeStruct((batch, dim), jnp.float32),
      scratch_types=[
          pltpu.VMEM((n_ch, CHUNK), jnp.int32),
          pltpu.VMEM((b_per_w, dim), jnp.float32),
          pltpu.SemaphoreType.DMA((n_ch,)),
          pltpu.SemaphoreType.DMA((n_ch,)),
      ],
  )
  def gather_kernel(idx_hbm, table_hbm, out_hbm, idx_v, rows_v, gsems, wsems):
    wid = lax.axis_index("s") * nc + lax.axis_index("c")
    pltpu.sync_copy(idx_hbm.at[pl.ds(wid * n_ch, n_ch)], idx_v)
    gathers = [
        pltpu.make_async_copy(
            table_hbm.at[idx_v.at[j]],
            rows_v.at[pl.ds(j * CHUNK, CHUNK)],
            gsems.at[j],
        )
        for j in range(n_ch)
    ]
    writes = [
        pltpu.make_async_copy(
            rows_v.at[pl.ds(j * CHUNK, CHUNK)],
            out_hbm.at[pl.ds(wid * b_per_w + j * CHUNK, CHUNK)],
            wsems.at[j],
        )
        for j in range(n_ch)
    ]
    for g in gathers:
      g.start()
    for j in range(n_ch):
      gathers[j].wait()
      writes[j].start()
    for w in writes:
      w.wait()

  return gather_kernel


def kernel(titles, embedding_table):
  batch = titles.shape[0]
  vocab, dim = embedding_table.shape
  idx2d = titles.astype(jnp.int32).reshape(batch // CHUNK, CHUNK)
  return _make_gather(batch, vocab, dim)(idx2d, embedding_table)


# speedup vs baseline: 1.0243x; 1.0243x over previous
import functools

import jax
import jax.numpy as jnp
from jax import lax
from jax.experimental import pallas as pl
from jax.experimental.pallas import tpu as pltpu
from jax.experimental.pallas import tpu_sc as plsc


@functools.lru_cache(maxsize=None)
def _make_gather(batch, vocab, dim):
  info = plsc.get_sparse_core_info()
  nc, ns = info.num_cores, info.num_subcores
  nw = nc * ns
  b_per_w = batch // nw
  mesh = plsc.VectorSubcoreMesh(core_axis_name="c", subcore_axis_name="s")

  @functools.partial(
      pl.kernel,
      mesh=mesh,
      out_type=jax.ShapeDtypeStruct((batch, dim), jnp.float32),
      scratch_types=[
          pltpu.VMEM((b_per_w,), jnp.int32),
          pltpu.VMEM((b_per_w, dim), jnp.float32),
          pltpu.SemaphoreType.DMA,
      ],
  )
  def gather_kernel(idx_hbm, table_hbm, out_hbm, idx_v, rows_v, sem):
    wid = lax.axis_index("s") * nc + lax.axis_index("c")
    base = wid * b_per_w
    pltpu.sync_copy(idx_hbm.at[pl.ds(base, b_per_w)], idx_v)
    pltpu.async_copy(table_hbm.at[idx_v], rows_v, sem).wait()
    pltpu.sync_copy(rows_v, out_hbm.at[pl.ds(base, b_per_w)])

  return gather_kernel


def kernel(titles, embedding_table):
  batch = titles.shape[0]
  vocab, dim = embedding_table.shape
  return _make_gather(batch, vocab, dim)(titles.astype(jnp.int32),
                                         embedding_table)
